# Initial kernel scaffold; baseline (speedup 1.0000x reference)
import jax
import jax.numpy as jnp
from jax.experimental import pallas as pl


def kernel(observations, next_observations, actions, rewards, dones, truncations, critic_observations, next_critic_observations, indices):
    raise NotImplementedError("write your pallas kernel here")


# speedup vs baseline: 3.1566x; 3.1566x over previous
import functools

import jax
import jax.numpy as jnp
from jax import lax
from jax.experimental import pallas as pl
from jax.experimental.pallas import tpu as pltpu
from jax.experimental.pallas import tpu_sc as plsc

N_ENV_ = 64
BUF_ = 2048
N_OBS_ = 128
N_ACT_ = 32
N_COBS_ = 160
BATCH_ = 1024

NC_, NS_ = 2, 16
NW_ = NC_ * NS_
EPW_ = N_ENV_ // NW_
RPW_ = EPW_ * BATCH_
SROWS_ = EPW_ * BUF_
CH_ = 128
NCHUNK_ = RPW_ // CH_
L_ = 16


def _sc_body(obs, nobs, acts, rew, dn, tr, cobs, ncobs, idx,
             obs_o, acts_o, rew_o, dn_o, tr_o, nobs_o, cobs_o, ncobs_o,
             idx_loc, idx_glob, rew_st, dn_st, tr_st, rew_ob, dn_ob, tr_ob,
             obs_b, nobs_b, acts_b, cobs_b, ncobs_b, gsem, wsem):
    wid = lax.axis_index("s") * NC_ + lax.axis_index("c")
    obase = wid * RPW_
    ibase = wid * SROWS_

    pltpu.sync_copy(idx.at[pl.ds(obase, RPW_)], idx_loc)
    pltpu.sync_copy(rew.at[pl.ds(ibase, SROWS_)], rew_st)
    pltpu.sync_copy(dn.at[pl.ds(ibase, SROWS_)], dn_st)
    pltpu.sync_copy(tr.at[pl.ds(ibase, SROWS_)], tr_st)

    def scal_body(g, carry):
        v = idx_loc[pl.ds(g * L_, L_)]
        v = v + (g // (BATCH_ // L_)) * BUF_
        idx_glob[pl.ds(g * L_, L_)] = v + ibase
        rew_ob[pl.ds(g * L_, L_)] = plsc.load_gather(rew_st, [v])
        dn_ob[pl.ds(g * L_, L_)] = plsc.load_gather(dn_st, [v])
        tr_ob[pl.ds(g * L_, L_)] = plsc.load_gather(tr_st, [v])
        return carry

    lax.fori_loop(0, RPW_ // L_, scal_body, 0)

    pltpu.sync_copy(rew_ob, rew_o.at[pl.ds(obase, RPW_)])
    pltpu.sync_copy(dn_ob, dn_o.at[pl.ds(obase, RPW_)])
    pltpu.sync_copy(tr_ob, tr_o.at[pl.ds(obase, RPW_)])

    tables = ((obs, obs_b, obs_o), (nobs, nobs_b, nobs_o),
              (acts, acts_b, acts_o), (cobs, cobs_b, cobs_o),
              (ncobs, ncobs_b, ncobs_o))

    def chunk_body(c, carry):
        idxr = idx_glob.at[pl.ds(c * CH_, CH_)]
        gd = [pltpu.make_async_copy(h.at[idxr], b, gsem) for (h, b, _) in tables]
        for d in gd:
            d.start()
        for d in gd:
            d.wait()
        ob = obase + c * CH_
        wd = [pltpu.make_async_copy(b, o.at[pl.ds(ob, CH_)], wsem)
              for (_, b, o) in tables]
        for d in wd:
            d.start()
        for d in wd:
            d.wait()
        return carry

    lax.fori_loop(0, NCHUNK_, chunk_body, 0)


def kernel(observations, next_observations, actions, rewards, dones,
           truncations, critic_observations, next_critic_observations,
           indices):
    nb = N_ENV_ * BATCH_
    v = N_ENV_ * BUF_
    f32, i32 = jnp.float32, dones.dtype

    mesh = plsc.VectorSubcoreMesh(core_axis_name="c", subcore_axis_name="s",
                                  num_cores=NC_, num_subcores=NS_)
    out_type = (
        jax.ShapeDtypeStruct((nb, N_OBS_), f32),
        jax.ShapeDtypeStruct((nb, N_ACT_), f32),
        jax.ShapeDtypeStruct((nb,), f32),
        jax.ShapeDtypeStruct((nb,), i32),
        jax.ShapeDtypeStruct((nb,), i32),
        jax.ShapeDtypeStruct((nb, N_OBS_), f32),
        jax.ShapeDtypeStruct((nb, N_COBS_), f32),
        jax.ShapeDtypeStruct((nb, N_COBS_), f32),
    )
    scratch = [
        pltpu.VMEM((RPW_,), jnp.int32),
        pltpu.VMEM((RPW_,), jnp.int32),
        pltpu.VMEM((SROWS_,), f32),
        pltpu.VMEM((SROWS_,), i32),
        pltpu.VMEM((SROWS_,), i32),
        pltpu.VMEM((RPW_,), f32),
        pltpu.VMEM((RPW_,), i32),
        pltpu.VMEM((RPW_,), i32),
        pltpu.VMEM((CH_, N_OBS_), f32),
        pltpu.VMEM((CH_, N_OBS_), f32),
        pltpu.VMEM((CH_, N_ACT_), f32),
        pltpu.VMEM((CH_, N_COBS_), f32),
        pltpu.VMEM((CH_, N_COBS_), f32),
        pltpu.SemaphoreType.DMA,
        pltpu.SemaphoreType.DMA,
    ]

    run = pl.kernel(_sc_body, out_type=out_type, mesh=mesh,
                    scratch_types=scratch,
                    compiler_params=pltpu.CompilerParams(
                        needs_layout_passes=False,
                        use_tc_tiling_on_sc=False))
    obs_o, acts_o, rew_o, dn_o, tr_o, nobs_o, cobs_o, ncobs_o = run(
        observations.reshape(v, N_OBS_),
        next_observations.reshape(v, N_OBS_),
        actions.reshape(v, N_ACT_),
        rewards.reshape(v),
        dones.reshape(v),
        truncations.reshape(v),
        critic_observations.reshape(v, N_COBS_),
        next_critic_observations.reshape(v, N_COBS_),
        indices.reshape(nb),
    )
    ens = jnp.ones((nb,), i32)
    return (obs_o, acts_o, rew_o, dn_o, tr_o, ens, nobs_o, cobs_o, ncobs_o)


# speedup vs baseline: 3.2739x; 1.0372x over previous
import jax
import jax.numpy as jnp
from jax import lax
from jax.experimental import pallas as pl
from jax.experimental.pallas import tpu as pltpu
from jax.experimental.pallas import tpu_sc as plsc

N_ENV_ = 64
BUF_ = 2048
N_OBS_ = 128
N_ACT_ = 32
N_COBS_ = 160
BATCH_ = 1024

NC_, NS_ = 2, 16
NW_ = NC_ * NS_
EPW_ = N_ENV_ // NW_
RPW_ = EPW_ * BATCH_
SROWS_ = EPW_ * BUF_
CH_ = 128
NCHUNK_ = RPW_ // CH_
NPAIR_ = NCHUNK_ // 2
L_ = 16


def _fill_indices(idx, idx_loc, idx_glob, ibase, extra=None):
    def body(g, carry):
        v = idx_loc[pl.ds(g * L_, L_)]
        v = v + (g // (BATCH_ // L_)) * BUF_
        idx_glob[pl.ds(g * L_, L_)] = v + ibase
        if extra is not None:
            extra(g, v)
        return carry

    lax.fori_loop(0, RPW_ // L_, body, 0)


def _pipe(tables, idx_glob, obase, g0, g1, w0, w1):
    def gstart(c, s, sem, wait=False):
        idxr = idx_glob.at[pl.ds(c * CH_, CH_)]
        for (h, bufs, _, _) in tables:
            dsc = pltpu.make_async_copy(h.at[idxr], bufs[s], sem)
            dsc.wait() if wait else dsc.start()

    def wstart(c, s, sem, wait=False):
        for (_, bufs, o, _) in tables:
            dsc = pltpu.make_async_copy(
                bufs[s], o.at[pl.ds(obase + c * CH_, CH_)], sem)
            dsc.wait() if wait else dsc.start()

    gstart(0, 0, g0)

    def pair(k, carry):
        a, b = 2 * k, 2 * k + 1

        @pl.when(k >= 1)
        def _():
            wstart(b - 2, 1, w1, wait=True)

        gstart(b, 1, g1)
        gstart(a, 0, g0, wait=True)
        wstart(a, 0, w0)

        @pl.when(k < NPAIR_ - 1)
        def _():
            wstart(a, 0, w0, wait=True)
            gstart(a + 2, 0, g0)

        gstart(b, 1, g1, wait=True)
        wstart(b, 1, w1)
        return carry

    lax.fori_loop(0, NPAIR_, pair, 0)
    wstart(NCHUNK_ - 2, 0, w0, wait=True)
    wstart(NCHUNK_ - 1, 1, w1, wait=True)


def _body_a(obs, nobs, rew, dn, tr, idx,
            obs_o, nobs_o, rew_o, dn_o, tr_o,
            idx_loc, idx_glob, rew_st, dn_st, tr_st, rew_ob, dn_ob, tr_ob,
            oa0, oa1, na0, na1, g0, g1, w0, w1):
    wid = lax.axis_index("s") * NC_ + lax.axis_index("c")
    obase = wid * RPW_
    ibase = wid * SROWS_

    pltpu.sync_copy(idx.at[pl.ds(obase, RPW_)], idx_loc)
    pltpu.sync_copy(rew.at[pl.ds(ibase, SROWS_)], rew_st)
    pltpu.sync_copy(dn.at[pl.ds(ibase, SROWS_)], dn_st)
    pltpu.sync_copy(tr.at[pl.ds(ibase, SROWS_)], tr_st)

    def scal(g, v):
        rew_ob[pl.ds(g * L_, L_)] = plsc.load_gather(rew_st, [v])
        dn_ob[pl.ds(g * L_, L_)] = plsc.load_gather(dn_st, [v])
        tr_ob[pl.ds(g * L_, L_)] = plsc.load_gather(tr_st, [v])

    _fill_indices(idx, idx_loc, idx_glob, ibase, extra=scal)

    pltpu.sync_copy(rew_ob, rew_o.at[pl.ds(obase, RPW_)])
    pltpu.sync_copy(dn_ob, dn_o.at[pl.ds(obase, RPW_)])
    pltpu.sync_copy(tr_ob, tr_o.at[pl.ds(obase, RPW_)])

    tables = ((obs, (oa0, oa1), obs_o, 128), (nobs, (na0, na1), nobs_o, 128))
    _pipe(tables, idx_glob, obase, g0, g1, w0, w1)


def _body_b(acts, cobs, ncobs, idx,
            acts_o, cobs_o, ncobs_o,
            idx_loc, idx_glob,
            ab0, ab1, cb0, cb1, nb0, nb1, g0, g1, w0, w1):
    wid = lax.axis_index("s") * NC_ + lax.axis_index("c")
    obase = wid * RPW_
    ibase = wid * SROWS_

    pltpu.sync_copy(idx.at[pl.ds(obase, RPW_)], idx_loc)
    _fill_indices(idx, idx_loc, idx_glob, ibase)

    tables = ((acts, (ab0, ab1), acts_o, N_ACT_),
              (cobs, (cb0, cb1), cobs_o, N_COBS_),
              (ncobs, (nb0, nb1), ncobs_o, N_COBS_))
    _pipe(tables, idx_glob, obase, g0, g1, w0, w1)


def kernel(observations, next_observations, actions, rewards, dones,
           truncations, critic_observations, next_critic_observations,
           indices):
    nb = N_ENV_ * BATCH_
    v = N_ENV_ * BUF_
    f32, i32 = jnp.float32, dones.dtype

    mesh = plsc.VectorSubcoreMesh(core_axis_name="c", subcore_axis_name="s",
                                  num_cores=NC_, num_subcores=NS_)
    params = pltpu.CompilerParams(needs_layout_passes=False,
                                  use_tc_tiling_on_sc=False)
    sems = [pltpu.SemaphoreType.DMA] * 4

    out_a = (
        jax.ShapeDtypeStruct((nb, N_OBS_), f32),
        jax.ShapeDtypeStruct((nb, N_OBS_), f32),
        jax.ShapeDtypeStruct((nb,), f32),
        jax.ShapeDtypeStruct((nb,), i32),
        jax.ShapeDtypeStruct((nb,), i32),
    )
    scratch_a = [
        pltpu.VMEM((RPW_,), jnp.int32),
        pltpu.VMEM((RPW_,), jnp.int32),
        pltpu.VMEM((SROWS_,), f32),
        pltpu.VMEM((SROWS_,), i32),
        pltpu.VMEM((SROWS_,), i32),
        pltpu.VMEM((RPW_,), f32),
        pltpu.VMEM((RPW_,), i32),
        pltpu.VMEM((RPW_,), i32),
        pltpu.VMEM((CH_, N_OBS_), f32),
        pltpu.VMEM((CH_, N_OBS_), f32),
        pltpu.VMEM((CH_, N_OBS_), f32),
        pltpu.VMEM((CH_, N_OBS_), f32),
    ] + sems

    out_b = (
        jax.ShapeDtypeStruct((nb, N_ACT_), f32),
        jax.ShapeDtypeStruct((nb, N_COBS_), f32),
        jax.ShapeDtypeStruct((nb, N_COBS_), f32),
    )
    scratch_b = [
        pltpu.VMEM((RPW_,), jnp.int32),
        pltpu.VMEM((RPW_,), jnp.int32),
        pltpu.VMEM((CH_, N_ACT_), f32),
        pltpu.VMEM((CH_, N_ACT_), f32),
        pltpu.VMEM((CH_, N_COBS_), f32),
        pltpu.VMEM((CH_, N_COBS_), f32),
        pltpu.VMEM((CH_, N_COBS_), f32),
        pltpu.VMEM((CH_, N_COBS_), f32),
    ] + sems

    idx_f = indices.reshape(nb)
    obs_o, nobs_o, rew_o, dn_o, tr_o = pl.kernel(
        _body_a, out_type=out_a, mesh=mesh, scratch_types=scratch_a,
        compiler_params=params)(
            observations.reshape(v, N_OBS_),
            next_observations.reshape(v, N_OBS_),
            rewards.reshape(v), dones.reshape(v), truncations.reshape(v),
            idx_f)
    acts_o, cobs_o, ncobs_o = pl.kernel(
        _body_b, out_type=out_b, mesh=mesh, scratch_types=scratch_b,
        compiler_params=params)(
            actions.reshape(v, N_ACT_),
            critic_observations.reshape(v, N_COBS_),
            next_critic_observations.reshape(v, N_COBS_),
            idx_f)

    ens = jnp.ones((nb,), i32)
    return (obs_o, acts_o, rew_o, dn_o, tr_o, ens, nobs_o, cobs_o, ncobs_o)
